# baseline (speedup 1.0000x reference)
import functools

import jax
import jax.numpy as jnp
from jax.experimental import pallas as pl
from jax.experimental.pallas import tpu as pltpu

D_MODEL = 768
NUM_EXPERTS = 64
TOP_K = 8
BLOCK_R = 6144


def _router_body(x_ref, w_ref, b_ref, logits_ref, rw_ref):
    lt = jax.lax.dot_general(
        w_ref[...], x_ref[...],
        dimension_numbers=(((0,), (1,)), ((), ())),
        preferred_element_type=jnp.float32,
    ) + b_ref[...]

    idx = jax.lax.broadcasted_iota(jnp.int32, lt.shape, 0)
    m = lt
    for _ in range(TOP_K):
        mx = jnp.max(m, axis=0, keepdims=True)
        eq = m == mx
        fi = jnp.min(jnp.where(eq, idx, NUM_EXPERTS), axis=0, keepdims=True)
        m = jnp.where(eq & (idx == fi), -jnp.inf, m)

    sel = m != lt
    q = jnp.minimum(jnp.floor(jnp.maximum(lt, 0.0) * 2.0) * 0.5, 7.5)
    rwt = jnp.where(sel, q, 0.0)
    logits_ref[...] = lt.T
    rw_ref[...] = rwt.T


@functools.partial(jax.jit, static_argnames=())
def kernel(x, W, b):
    n_tokens = x.shape[0]
    grid = ((n_tokens + BLOCK_R - 1) // BLOCK_R,)
    logits, rw = pl.pallas_call(
        _router_body,
        grid=grid,
        in_specs=[
            pl.BlockSpec((BLOCK_R, D_MODEL), lambda i: (i, 0)),
            pl.BlockSpec((D_MODEL, NUM_EXPERTS), lambda i: (0, 0)),
            pl.BlockSpec((NUM_EXPERTS, 1), lambda i: (0, 0)),
        ],
        out_specs=[
            pl.BlockSpec((BLOCK_R, NUM_EXPERTS), lambda i: (i, 0)),
            pl.BlockSpec((BLOCK_R, NUM_EXPERTS), lambda i: (i, 0)),
        ],
        out_shape=[
            jax.ShapeDtypeStruct((n_tokens, NUM_EXPERTS), jnp.float32),
            jax.ShapeDtypeStruct((n_tokens, NUM_EXPERTS), jnp.float32),
        ],
        compiler_params=pltpu.CompilerParams(
            dimension_semantics=("parallel",),
        ),
    )(x, W, b.reshape(NUM_EXPERTS, 1))
    return (rw, logits)


# speedup vs baseline: 1.0731x; 1.0731x over previous
import functools

import jax
import jax.numpy as jnp
from jax.experimental import pallas as pl
from jax.experimental.pallas import tpu as pltpu

D_MODEL = 768
NUM_EXPERTS = 64
TOP_K = 8
BLOCK_R = 4096


def _router_body(x_ref, w_ref, b_ref, logits_ref, rw_ref):
    lt = jax.lax.dot_general(
        w_ref[...], x_ref[...],
        dimension_numbers=(((0,), (1,)), ((), ())),
        preferred_element_type=jnp.float32,
    ) + b_ref[...]

    idx = jax.lax.broadcasted_iota(jnp.int32, lt.shape, 0)
    m = lt
    for _ in range(TOP_K):
        mx = jnp.max(m, axis=0, keepdims=True)
        eq = m == mx
        fi = jnp.min(jnp.where(eq, idx, NUM_EXPERTS), axis=0, keepdims=True)
        m = jnp.where(eq & (idx == fi), -jnp.inf, m)

    sel = m != lt
    q = jnp.minimum(jnp.floor(jnp.maximum(lt, 0.0) * 2.0) * 0.5, 7.5)
    rwt = jnp.where(sel, q, 0.0)
    logits_ref[...] = lt.T
    rw_ref[...] = rwt.T


@functools.partial(jax.jit, static_argnames=())
def kernel(x, W, b):
    n_tokens = x.shape[0]
    grid = (n_tokens // BLOCK_R,)
    logits, rw = pl.pallas_call(
        _router_body,
        grid=grid,
        in_specs=[
            pl.BlockSpec((BLOCK_R, D_MODEL), lambda i: (i, 0)),
            pl.BlockSpec((D_MODEL, NUM_EXPERTS), lambda i: (0, 0)),
            pl.BlockSpec((NUM_EXPERTS, 1), lambda i: (0, 0)),
        ],
        out_specs=[
            pl.BlockSpec((BLOCK_R, NUM_EXPERTS), lambda i: (i, 0)),
            pl.BlockSpec((BLOCK_R, NUM_EXPERTS), lambda i: (i, 0)),
        ],
        out_shape=[
            jax.ShapeDtypeStruct((n_tokens, NUM_EXPERTS), jnp.float32),
            jax.ShapeDtypeStruct((n_tokens, NUM_EXPERTS), jnp.float32),
        ],
        compiler_params=pltpu.CompilerParams(
            dimension_semantics=("parallel",),
        ),
    )(x, W, b.reshape(NUM_EXPERTS, 1))
    return (rw, logits)
